# baseline (speedup 1.0000x reference)
import functools

import jax
import jax.numpy as jnp
from jax import lax
from jax.experimental import pallas as pl
from jax.experimental.pallas import tpu as pltpu

KB = 64
LN = 128
TB = 8


def _topk_body(tgt_ref, u_ref, vals_ref, idx_ref, x_ref, *, rs, nprog_t):
    pid = pl.program_id(0)
    b = pid // nprog_t
    tj = pid % nprog_t
    sub = lax.broadcasted_iota(jnp.int32, (rs, LN), 0)
    lane = lax.broadcasted_iota(jnp.int32, (rs, LN), 1)
    gidx = sub * LN + lane
    lane128 = lax.broadcasted_iota(jnp.int32, (1, LN), 1)
    lane64 = lax.broadcasted_iota(jnp.int32, (1, KB), 1)
    iota_c = lax.broadcasted_iota(jnp.int32, (TB, rs), 1)
    iota_r = lax.broadcasted_iota(jnp.int32, (TB, rs), 0)
    r64 = lax.broadcasted_iota(jnp.int32, (TB, KB), 0)
    c64 = lax.broadcasted_iota(jnp.int32, (TB, KB), 1)
    BIG = jnp.int32(2**30)

    golds = []
    rowmax = jnp.zeros((TB, rs), jnp.float32)
    for r in range(TB):
        t = tj * TB + r
        tgt = tgt_ref[b, t]
        x0 = u_ref[0, r]
        is_t = gidx == tgt
        golds.append(jnp.sum(jnp.where(is_t, x0, 0.0)))
        x = jnp.where(is_t, jnp.inf, x0)
        x_ref[r] = x
        rm = jnp.max(x, axis=1).reshape(1, rs)
        rowmax = jnp.where(iota_r == r, rm, rowmax)

    def body(k, c):
        rowmax, vals, ids = c
        for r in range(TB):
            rm = rowmax[r:r + 1, :]
            m = jnp.max(rm)
            cid = jnp.min(jnp.where(rm == m, iota_c[r:r + 1, :], BIG))
            xr = x_ref[r, pl.ds(cid, 1), :]
            l = jnp.min(jnp.where(xr == m, lane128, BIG))
            gi = cid * LN + l
            xr2 = jnp.where(lane128 == l, -jnp.inf, xr)
            x_ref[r, pl.ds(cid, 1), :] = xr2
            nm = jnp.max(xr2)
            rowmax = jnp.where((iota_r == r) & (iota_c == cid), nm, rowmax)
            vals = jnp.where((r64 == r) & (c64 == k), m, vals)
            ids = jnp.where((r64 == r) & (c64 == k), gi, ids)
        return rowmax, vals, ids

    _, vals, ids = lax.fori_loop(
        0, KB, body,
        (rowmax, jnp.zeros((TB, KB), jnp.float32), jnp.zeros((TB, KB), jnp.int32)))
    for r in range(TB):
        vals = jnp.where((r64 == r) & (c64 == 0), golds[r], vals)
    vals_ref[0] = vals
    idx_ref[0] = ids


def _tail_body(mask_smem, vals_ref, g1_ref, g2_ref, mask_ref, out_ref, *, t):
    b = pl.program_id(0)

    @pl.when(b == 0)
    def _():
        out_ref[0, 0] = 0.0

    v = vals_ref[0]
    mx = jnp.max(v, axis=-1, keepdims=True)
    lse = jnp.log(jnp.sum(jnp.exp(v - mx), axis=-1, keepdims=True)) + mx
    node = (v[:, :1] - lse).reshape(1, t)
    msk = mask_ref[0, 0].reshape(1, t)
    node_sum = jnp.sum(node * msk)
    tsum = jnp.sum(msk)

    r64 = lax.broadcasted_iota(jnp.int32, (KB, KB), 0)
    c64 = lax.broadcasted_iota(jnp.int32, (KB, KB), 1)

    def body(i, acc):
        s1 = g1_ref[0, i]
        s2 = g2_ref[0, i + 1]
        m = lax.dot_general(s1, s2, (((1,), (1,)), ((), ())),
                            preferred_element_type=jnp.float32)
        m00 = jnp.sum(jnp.where((r64 == 0) & (c64 == 0), m, 0.0))
        mmx = jnp.max(m)
        else_ = jnp.log(jnp.sum(jnp.exp(m - mmx))) + mmx
        return acc + (m00 - else_) * mask_smem[b, i + 1].astype(jnp.float32)

    edge_sum = lax.fori_loop(0, t - 1, body, jnp.float32(0.0))
    nb = pl.num_programs(0)
    out_ref[0, 0] += -(node_sum + edge_sum) / tsum / nb


def kernel(unaries, masks, targets, E1_weight, E2_weight):
    b, t, v = unaries.shape
    rs = v // LN
    nprog_t = t // TB
    nprog = b * nprog_t
    u4 = unaries.reshape(nprog, TB, rs, LN)

    vals, ids = pl.pallas_call(
        functools.partial(_topk_body, rs=rs, nprog_t=nprog_t),
        grid=(nprog,),
        in_specs=[
            pl.BlockSpec(memory_space=pltpu.SMEM),
            pl.BlockSpec((1, TB, rs, LN), lambda i: (i, 0, 0, 0)),
        ],
        out_specs=[
            pl.BlockSpec((1, TB, KB), lambda i: (i, 0, 0)),
            pl.BlockSpec((1, TB, KB), lambda i: (i, 0, 0)),
        ],
        out_shape=[
            jax.ShapeDtypeStruct((nprog, TB, KB), jnp.float32),
            jax.ShapeDtypeStruct((nprog, TB, KB), jnp.int32),
        ],
        scratch_shapes=[pltpu.VMEM((TB, rs, LN), jnp.float32)],
    )(targets.astype(jnp.int32), u4)

    vals = vals.reshape(b, t, KB)
    ids = ids.reshape(b, t, KB)

    g1 = jnp.take(E1_weight, ids, axis=0)
    g2 = jnp.take(E2_weight, ids, axis=0)

    nll = pl.pallas_call(
        functools.partial(_tail_body, t=t),
        grid=(b,),
        in_specs=[
            pl.BlockSpec(memory_space=pltpu.SMEM),
            pl.BlockSpec((1, t, KB), lambda i: (i, 0, 0)),
            pl.BlockSpec((1, t, KB, g1.shape[-1]), lambda i: (i, 0, 0, 0)),
            pl.BlockSpec((1, t, KB, g1.shape[-1]), lambda i: (i, 0, 0, 0)),
            pl.BlockSpec((1, 1, t), lambda i: (i, 0, 0)),
        ],
        out_specs=pl.BlockSpec(memory_space=pltpu.SMEM),
        out_shape=jax.ShapeDtypeStruct((1, 1), jnp.float32),
    )(masks.astype(jnp.int32), vals, g1, g2,
      masks.astype(jnp.float32).reshape(b, 1, t))

    return nll[0, 0]


# speedup vs baseline: 9.5651x; 9.5651x over previous
import functools

import jax
import jax.numpy as jnp
from jax import lax
from jax.experimental import pallas as pl
from jax.experimental.pallas import tpu as pltpu
from jax.experimental.pallas import tpu_sc as plsc

KB = 64
LN = 128
TB = 8
NW = 32
SL = 16


def _scal(x):
    return jnp.max(x, axis=0) if x.ndim else x


def _tau_body(u_ref, tau_ref, *, rs):
    riota = lax.broadcasted_iota(jnp.int32, (TB, LN), 0)
    lane = lax.broadcasted_iota(jnp.int32, (TB, LN), 1)
    cm = jnp.full((TB, LN), -jnp.inf, jnp.float32)
    for r in range(TB):
        x0 = u_ref[0, r]
        c = jnp.max(x0, axis=0).reshape(1, LN)
        cm = jnp.where(riota == r, c, cm)

    def body(k, carry):
        cmx, _ = carry
        m8 = jnp.max(cmx, axis=1, keepdims=True)
        am = jnp.min(jnp.where(cmx == m8, lane, jnp.int32(2**30)),
                     axis=1, keepdims=True)
        cmx = jnp.where(lane == am, -jnp.inf, cmx)
        return cmx, m8

    _, tau8 = lax.fori_loop(0, KB, body,
                            (cm, jnp.zeros((TB, 1), jnp.float32)))
    tau_ref[0] = tau8.reshape(1, TB)


def _sc_beam_body(u_hbm, tgt_hbm, tau_hbm, e1_hbm, e2_hbm,
                  vals_hbm, g1_hbm, g2_hbm,
                  row_v, cval_v, cidx_v, tgts_v, taus_v, vals_v, ids_v,
                  g1_v, g2_v, gpad_v, sem, *, v, rpw, d):
    wid = lax.axis_index("s") * 2 + lax.axis_index("c")
    base = wid * rpw
    pltpu.sync_copy(tgt_hbm.at[pl.ds(base, rpw)], tgts_v)
    pltpu.sync_copy(tau_hbm.at[pl.ds(base, rpw)], taus_v)
    iota16 = lax.iota(jnp.int32, SL)
    NINF = jnp.float32(-jnp.inf)
    ninf_v = jnp.full((SL,), NINF)
    zero_i = jnp.zeros((SL,), jnp.int32)
    zero_f = jnp.zeros((SL,), jnp.float32)

    nsub = LN // SL

    def do_row(i, _):
        row = base + i
        pltpu.sync_copy(u_hbm.at[row], row_v)
        bsel = zero_i + i
        tgtv = plsc.load_gather(tgts_v, [bsel])
        tauv = plsc.load_gather(taus_v, [bsel])
        goldv = plsc.load_gather(row_v, [tgtv // LN, tgtv % LN])

        def filt(j, off):
            xv = row_v[j // nsub, pl.ds((j % nsub) * SL, SL)]
            iv = j * SL + iota16
            m = (xv >= tauv) & (iv != tgtv)
            cnt = _scal(plsc.all_reduce_population_count(m))

            def hit(o):
                plsc.store_compressed(cval_v.at[pl.ds(o, SL)], xv, mask=m)
                plsc.store_compressed(cidx_v.at[pl.ds(o, SL)], iv, mask=m)
                return o + cnt

            return lax.cond(cnt > 0, hit, lambda o: o, off)

        c = lax.fori_loop(0, v // SL, filt, jnp.int32(0))
        cval_v[pl.ds(c, SL)] = ninf_v
        c16 = (c + SL - 1) // SL

        a0v = jnp.where(iota16 == 0, goldv, ninf_v)
        a0i = jnp.where(iota16 == 0, tgtv, zero_i)

        def ext(k, carry):
            avs = list(carry[:4])
            ais = list(carry[4:])

            def scan(j, bc):
                best, pos = bc
                cv = cval_v[pl.ds(j * SL, SL)]
                mj = jnp.max(cv, axis=0)
                pj = _scal(plsc.all_reduce_ffs(cv == (zero_f + mj)))
                upd = mj > best
                return (jnp.where(upd, mj, best),
                        jnp.where(upd, j * SL + pj, pos))

            best, pos = lax.fori_loop(0, c16, scan, (NINF, jnp.int32(0)))
            posv = zero_i + pos
            bvv = zero_f + best
            biv = plsc.load_gather(cidx_v, [posv])
            plsc.store_scatter(cval_v, [posv], ninf_v, mask=iota16 == 0)
            km = k % SL
            kd = k // SL
            for j in range(4):
                m = (iota16 == km) & (kd == j)
                avs[j] = jnp.where(m, bvv, avs[j])
                ais[j] = jnp.where(m, biv, ais[j])
            return tuple(avs) + tuple(ais)

        accs = lax.fori_loop(
            1, KB, ext,
            (a0v, ninf_v, ninf_v, ninf_v, a0i, zero_i, zero_i, zero_i))
        for j in range(4):
            vals_v[pl.ds(j * SL, SL)] = accs[j]
            ids_v[pl.ds(j * SL, SL)] = accs[4 + j]
        for ghbm, gv in ((e1_hbm, g1_v), (e2_hbm, g2_v)):
            pltpu.async_copy(ghbm.at[ids_v], gpad_v, sem).wait()

            def compact(kk, _, gv=gv):
                for jj in range(d // SL):
                    gv[kk, pl.ds(jj * SL, SL)] = gpad_v[kk, pl.ds(jj * SL, SL)]
                return 0

            lax.fori_loop(0, KB, compact, 0)
        pltpu.sync_copy(vals_v, vals_hbm.at[row])
        pltpu.sync_copy(g1_v, g1_hbm.at[row])
        pltpu.sync_copy(g2_v, g2_hbm.at[row])
        return 0

    lax.fori_loop(0, rpw, do_row, 0)


def _tail_body(mask_smem, vals_ref, g1_ref, g2_ref, mask_ref, out_ref, *, t):
    b = pl.program_id(0)

    @pl.when(b == 0)
    def _():
        out_ref[0, 0] = 0.0

    v = vals_ref[0]
    mx = jnp.max(v, axis=-1, keepdims=True)
    lse = jnp.log(jnp.sum(jnp.exp(v - mx), axis=-1, keepdims=True)) + mx
    node = (v[:, :1] - lse).reshape(1, t)
    msk = mask_ref[0, 0].reshape(1, t)
    node_sum = jnp.sum(node * msk)
    tsum = jnp.sum(msk)

    r64 = lax.broadcasted_iota(jnp.int32, (KB, KB), 0)
    c64 = lax.broadcasted_iota(jnp.int32, (KB, KB), 1)

    def body(i, acc):
        s1 = g1_ref[0, i]
        s2 = g2_ref[0, i + 1]
        m = lax.dot_general(s1, s2, (((1,), (1,)), ((), ())),
                            preferred_element_type=jnp.float32)
        m00 = jnp.sum(jnp.where((r64 == 0) & (c64 == 0), m, 0.0))
        mmx = jnp.max(m)
        else_ = jnp.log(jnp.sum(jnp.exp(m - mmx))) + mmx
        return acc + (m00 - else_) * mask_smem[b, i + 1].astype(jnp.float32)

    edge_sum = lax.fori_loop(0, t - 1, body, jnp.float32(0.0))
    nb = pl.num_programs(0)
    out_ref[0, 0] += -(node_sum + edge_sum) / tsum / nb


def kernel(unaries, masks, targets, E1_weight, E2_weight):
    b, t, v = unaries.shape
    d = E1_weight.shape[-1]
    rs = v // LN
    nrows = b * t
    nprog = nrows // TB
    u4 = unaries.reshape(nprog, TB, rs, LN)
    u3 = unaries.reshape(nrows, rs, LN)

    tau3 = pl.pallas_call(
        functools.partial(_tau_body, rs=rs),
        grid=(nprog,),
        in_specs=[pl.BlockSpec((1, TB, rs, LN), lambda i: (i, 0, 0, 0))],
        out_specs=pl.BlockSpec((1, 1, TB), lambda i: (i, 0, 0)),
        out_shape=jax.ShapeDtypeStruct((nprog, 1, TB), jnp.float32),
    )(u4)
    tau = tau3.reshape(nrows)

    rpw = nrows // NW
    sc_beam = functools.partial(
        pl.kernel,
        out_type=[
            jax.ShapeDtypeStruct((nrows, KB), jnp.float32),
            jax.ShapeDtypeStruct((nrows, KB, d), jnp.float32),
            jax.ShapeDtypeStruct((nrows, KB, d), jnp.float32),
        ],
        mesh=plsc.VectorSubcoreMesh(core_axis_name="c", subcore_axis_name="s",
                                    num_cores=2, num_subcores=16),
        compiler_params=pltpu.CompilerParams(needs_layout_passes=False),
        scratch_types=[
            pltpu.VMEM((v // LN, LN), jnp.float32),
            pltpu.VMEM((v + SL,), jnp.float32),
            pltpu.VMEM((v + SL,), jnp.int32),
            pltpu.VMEM((rpw,), jnp.int32),
            pltpu.VMEM((rpw,), jnp.float32),
            pltpu.VMEM((KB,), jnp.float32),
            pltpu.VMEM((KB,), jnp.int32),
            pltpu.VMEM((KB, d), jnp.float32),
            pltpu.VMEM((KB, d), jnp.float32),
            pltpu.VMEM((KB, LN), jnp.float32),
            pltpu.SemaphoreType.DMA,
        ],
    )(functools.partial(_sc_beam_body, v=v, rpw=rpw, d=d))
    e1p = jnp.pad(E1_weight, ((0, 0), (0, LN - d)))
    e2p = jnp.pad(E2_weight, ((0, 0), (0, LN - d)))
    vals, g1, g2 = sc_beam(u3, targets.astype(jnp.int32).reshape(nrows), tau,
                           e1p, e2p)

    vals = vals.reshape(b, t, KB)
    g1 = g1.reshape(b, t, KB, d)
    g2 = g2.reshape(b, t, KB, d)

    nll = pl.pallas_call(
        functools.partial(_tail_body, t=t),
        grid=(b,),
        in_specs=[
            pl.BlockSpec(memory_space=pltpu.SMEM),
            pl.BlockSpec((1, t, KB), lambda i: (i, 0, 0)),
            pl.BlockSpec((1, t, KB, d), lambda i: (i, 0, 0, 0)),
            pl.BlockSpec((1, t, KB, d), lambda i: (i, 0, 0, 0)),
            pl.BlockSpec((1, 1, t), lambda i: (i, 0, 0)),
        ],
        out_specs=pl.BlockSpec(memory_space=pltpu.SMEM),
        out_shape=jax.ShapeDtypeStruct((1, 1), jnp.float32),
    )(masks.astype(jnp.int32), vals, g1, g2,
      masks.astype(jnp.float32).reshape(b, 1, t))

    return nll[0, 0]


# speedup vs baseline: 10.5511x; 1.1031x over previous
import functools

import jax
import jax.numpy as jnp
from jax import lax
from jax.experimental import pallas as pl
from jax.experimental.pallas import tpu as pltpu
from jax.experimental.pallas import tpu_sc as plsc

KB = 64
LN = 128
TB = 8
NW = 32
SL = 16


def _scal(x):
    return jnp.max(x, axis=0) if x.ndim else x


def _tau_body(u_ref, tau_ref, *, rs):
    riota = lax.broadcasted_iota(jnp.int32, (TB, LN), 0)
    lane = lax.broadcasted_iota(jnp.int32, (TB, LN), 1)
    cm = jnp.full((TB, LN), -jnp.inf, jnp.float32)
    for r in range(TB):
        x0 = u_ref[r]
        c = jnp.max(x0, axis=0).reshape(1, LN)
        cm = jnp.where(riota == r, c, cm)

    def body(k, carry):
        cmx, _ = carry
        m8 = jnp.max(cmx, axis=1, keepdims=True)
        am = jnp.min(jnp.where(cmx == m8, lane, jnp.int32(2**30)),
                     axis=1, keepdims=True)
        cmx = jnp.where(lane == am, -jnp.inf, cmx)
        return cmx, m8

    _, tau8 = lax.fori_loop(0, KB, body,
                            (cm, jnp.zeros((TB, 1), jnp.float32)))
    tau_ref[0] = tau8.reshape(1, TB)


def _sc_beam_body(u_hbm, tgt_hbm, tau_hbm, e1_hbm, e2_hbm,
                  vals_hbm, g1_hbm, g2_hbm,
                  row_v, cval_v, cidx_v, tgts_v, taus_v, vals_v, ids_v,
                  g1_v, g2_v, gpad_v, sem, *, v, rpw, d):
    wid = lax.axis_index("s") * 2 + lax.axis_index("c")
    base = wid * rpw
    pltpu.sync_copy(tgt_hbm.at[pl.ds(base, rpw)], tgts_v)
    pltpu.sync_copy(tau_hbm.at[pl.ds(base, rpw)], taus_v)
    iota16 = lax.iota(jnp.int32, SL)
    NINF = jnp.float32(-jnp.inf)
    ninf_v = jnp.full((SL,), NINF)
    zero_i = jnp.zeros((SL,), jnp.int32)
    zero_f = jnp.zeros((SL,), jnp.float32)

    nsub = LN // SL

    def do_row(i, _):
        row = base + i
        pltpu.sync_copy(u_hbm.at[row], row_v)
        bsel = zero_i + i
        tgtv = plsc.load_gather(tgts_v, [bsel])
        tauv = plsc.load_gather(taus_v, [bsel])
        goldv = plsc.load_gather(row_v, [tgtv // LN, tgtv % LN])

        def filt(j, off):
            xv = row_v[j // nsub, pl.ds((j % nsub) * SL, SL)]
            iv = j * SL + iota16
            m = (xv >= tauv) & (iv != tgtv)
            cnt = _scal(plsc.all_reduce_population_count(m))

            def hit(o):
                plsc.store_compressed(cval_v.at[pl.ds(o, SL)], xv, mask=m)
                plsc.store_compressed(cidx_v.at[pl.ds(o, SL)], iv, mask=m)
                return o + cnt

            return lax.cond(cnt > 0, hit, lambda o: o, off)

        c = lax.fori_loop(0, v // SL, filt, jnp.int32(0))
        cval_v[pl.ds(c, SL)] = ninf_v
        c16 = (c + SL - 1) // SL

        a0v = jnp.where(iota16 == 0, goldv, ninf_v)
        a0i = jnp.where(iota16 == 0, tgtv, zero_i)

        def ext(k, carry):
            avs = list(carry[:4])
            ais = list(carry[4:])

            def scan(j, bc):
                best, pos = bc
                cv = cval_v[pl.ds(j * SL, SL)]
                mj = jnp.max(cv, axis=0)
                pj = _scal(plsc.all_reduce_ffs(cv == (zero_f + mj)))
                upd = mj > best
                return (jnp.where(upd, mj, best),
                        jnp.where(upd, j * SL + pj, pos))

            best, pos = lax.fori_loop(0, c16, scan, (NINF, jnp.int32(0)))
            posv = zero_i + pos
            bvv = zero_f + best
            biv = plsc.load_gather(cidx_v, [posv])
            plsc.store_scatter(cval_v, [posv], ninf_v, mask=iota16 == 0)
            km = k % SL
            kd = k // SL
            for j in range(4):
                m = (iota16 == km) & (kd == j)
                avs[j] = jnp.where(m, bvv, avs[j])
                ais[j] = jnp.where(m, biv, ais[j])
            return tuple(avs) + tuple(ais)

        accs = lax.fori_loop(
            1, KB, ext,
            (a0v, ninf_v, ninf_v, ninf_v, a0i, zero_i, zero_i, zero_i))
        for j in range(4):
            vals_v[pl.ds(j * SL, SL)] = accs[j]
            ids_v[pl.ds(j * SL, SL)] = accs[4 + j]
        for ghbm, gv in ((e1_hbm, g1_v), (e2_hbm, g2_v)):
            pltpu.async_copy(ghbm.at[ids_v], gpad_v, sem).wait()

            def compact(kk, _, gv=gv):
                for jj in range(d // SL):
                    gv[kk, pl.ds(jj * SL, SL)] = gpad_v[kk, pl.ds(jj * SL, SL)]
                return 0

            lax.fori_loop(0, KB, compact, 0)
        pltpu.sync_copy(vals_v, vals_hbm.at[row])
        pltpu.sync_copy(g1_v, g1_hbm.at[row])
        pltpu.sync_copy(g2_v, g2_hbm.at[row])
        return 0

    lax.fori_loop(0, rpw, do_row, 0)


def _tail_body(mask_smem, vals_ref, g1_ref, g2_ref, mask_ref, out_ref, *, t):
    b = pl.program_id(0)

    @pl.when(b == 0)
    def _():
        out_ref[0, 0] = 0.0

    v = vals_ref[0]
    mx = jnp.max(v, axis=-1, keepdims=True)
    lse = jnp.log(jnp.sum(jnp.exp(v - mx), axis=-1, keepdims=True)) + mx
    node = (v[:, :1] - lse).reshape(1, t)
    msk = mask_ref[0, 0].reshape(1, t)
    node_sum = jnp.sum(node * msk)
    tsum = jnp.sum(msk)

    s1 = g1_ref[0, :t - 1]
    s2 = g2_ref[0, 1:]
    mm = lax.dot_general(s1, s2, (((2,), (2,)), ((0,), (0,))),
                         preferred_element_type=jnp.float32)
    mf = mm.reshape(t - 1, KB * KB)
    mx2 = jnp.max(mf, axis=-1, keepdims=True)
    lse2 = jnp.log(jnp.sum(jnp.exp(mf - mx2), axis=-1, keepdims=True)) + mx2
    edge = (mf[:, :1] - lse2).reshape(1, t - 1)
    edge_sum = jnp.sum(edge * msk[:, 1:])
    nb = pl.num_programs(0)
    out_ref[0, 0] += -(node_sum + edge_sum) / tsum / nb


def kernel(unaries, masks, targets, E1_weight, E2_weight):
    b, t, v = unaries.shape
    d = E1_weight.shape[-1]
    rs = v // LN
    nrows = b * t
    nprog = nrows // TB
    u3 = unaries.reshape(nrows, rs, LN)

    tau3 = pl.pallas_call(
        functools.partial(_tau_body, rs=rs),
        grid=(nprog,),
        in_specs=[pl.BlockSpec((TB, rs, LN), lambda i: (i, 0, 0))],
        out_specs=pl.BlockSpec((1, 1, TB), lambda i: (i, 0, 0)),
        out_shape=jax.ShapeDtypeStruct((nprog, 1, TB), jnp.float32),
    )(u3)
    tau = tau3.reshape(nrows)

    rpw = nrows // NW
    sc_beam = functools.partial(
        pl.kernel,
        out_type=[
            jax.ShapeDtypeStruct((nrows, KB), jnp.float32),
            jax.ShapeDtypeStruct((nrows, KB, d), jnp.float32),
            jax.ShapeDtypeStruct((nrows, KB, d), jnp.float32),
        ],
        mesh=plsc.VectorSubcoreMesh(core_axis_name="c", subcore_axis_name="s",
                                    num_cores=2, num_subcores=16),
        compiler_params=pltpu.CompilerParams(needs_layout_passes=False),
        scratch_types=[
            pltpu.VMEM((v // LN, LN), jnp.float32),
            pltpu.VMEM((v + SL,), jnp.float32),
            pltpu.VMEM((v + SL,), jnp.int32),
            pltpu.VMEM((rpw,), jnp.int32),
            pltpu.VMEM((rpw,), jnp.float32),
            pltpu.VMEM((KB,), jnp.float32),
            pltpu.VMEM((KB,), jnp.int32),
            pltpu.VMEM((KB, d), jnp.float32),
            pltpu.VMEM((KB, d), jnp.float32),
            pltpu.VMEM((KB, LN), jnp.float32),
            pltpu.SemaphoreType.DMA,
        ],
    )(functools.partial(_sc_beam_body, v=v, rpw=rpw, d=d))
    e1p = jnp.pad(E1_weight, ((0, 0), (0, LN - d)))
    e2p = jnp.pad(E2_weight, ((0, 0), (0, LN - d)))
    vals, g1, g2 = sc_beam(u3, targets.astype(jnp.int32).reshape(nrows), tau,
                           e1p, e2p)

    vals = vals.reshape(b, t, KB)
    g1 = g1.reshape(b, t, KB, d)
    g2 = g2.reshape(b, t, KB, d)

    nll = pl.pallas_call(
        functools.partial(_tail_body, t=t),
        grid=(b,),
        in_specs=[
            pl.BlockSpec(memory_space=pltpu.SMEM),
            pl.BlockSpec((1, t, KB), lambda i: (i, 0, 0)),
            pl.BlockSpec((1, t, KB, d), lambda i: (i, 0, 0, 0)),
            pl.BlockSpec((1, t, KB, d), lambda i: (i, 0, 0, 0)),
            pl.BlockSpec((1, 1, t), lambda i: (i, 0, 0)),
        ],
        out_specs=pl.BlockSpec(memory_space=pltpu.SMEM),
        out_shape=jax.ShapeDtypeStruct((1, 1), jnp.float32),
    )(masks.astype(jnp.int32), vals, g1, g2,
      masks.astype(jnp.float32).reshape(b, 1, t))

    return nll[0, 0]


# speedup vs baseline: 14.1849x; 1.3444x over previous
import functools

import jax
import jax.numpy as jnp
from jax import lax
from jax.experimental import pallas as pl
from jax.experimental.pallas import tpu as pltpu
from jax.experimental.pallas import tpu_sc as plsc

KB = 64
LN = 128
TB = 8
NW = 32
SL = 16


def _scal(x):
    return jnp.max(x, axis=0) if x.ndim else x


def _tau_body(u_ref, tau_ref, *, rs):
    riota = lax.broadcasted_iota(jnp.int32, (TB, LN), 0)
    lane = lax.broadcasted_iota(jnp.int32, (TB, LN), 1)
    cm = jnp.full((TB, LN), -jnp.inf, jnp.float32)
    for r in range(TB):
        x0 = u_ref[r]
        c = jnp.max(x0, axis=0).reshape(1, LN)
        cm = jnp.where(riota == r, c, cm)

    def body(k, carry):
        cmx, _ = carry
        m8 = jnp.max(cmx, axis=1, keepdims=True)
        am = jnp.min(jnp.where(cmx == m8, lane, jnp.int32(2**30)),
                     axis=1, keepdims=True)
        cmx = jnp.where(lane == am, -jnp.inf, cmx)
        return cmx, m8

    _, tau8 = lax.fori_loop(0, KB, body,
                            (cm, jnp.zeros((TB, 1), jnp.float32)))
    tau_ref[0] = tau8.reshape(1, TB)


def _sc_beam_body(u_hbm, tgt_hbm, tau_hbm, e1_hbm, e2_hbm,
                  vals_hbm, g1_hbm, g2_hbm,
                  row_v, cval_v, cidx_v, tgts_v, taus_v, vals_v, ids_v,
                  g1_v, g2_v, gpad_v, sem, *, v, rpw, d):
    wid = lax.axis_index("s") * 2 + lax.axis_index("c")
    base = wid * rpw
    pltpu.sync_copy(tgt_hbm.at[pl.ds(base, rpw)], tgts_v)
    pltpu.sync_copy(tau_hbm.at[pl.ds(base, rpw)], taus_v)
    iota16 = lax.iota(jnp.int32, SL)
    NINF = jnp.float32(-jnp.inf)
    ninf_v = jnp.full((SL,), NINF)
    zero_i = jnp.zeros((SL,), jnp.int32)
    zero_f = jnp.zeros((SL,), jnp.float32)

    nsub = LN // SL

    def do_row(i, _):
        row = base + i
        pltpu.sync_copy(u_hbm.at[row], row_v)
        bsel = zero_i + i
        tgtv = plsc.load_gather(tgts_v, [bsel])
        tauv = plsc.load_gather(taus_v, [bsel])
        goldv = plsc.load_gather(row_v, [tgtv // LN, tgtv % LN])

        taus = jnp.max(tauv, axis=0)

        def filt(jr, off):
            bm = row_v[jr, pl.ds(0, SL)]
            for js in range(1, nsub):
                bm = jnp.maximum(bm, row_v[jr, pl.ds(js * SL, SL)])
            bms = jnp.max(bm, axis=0)

            def hit(o):
                def sub(js, oo):
                    xv = row_v[jr, pl.ds(js * SL, SL)]
                    iv = jr * LN + js * SL + iota16
                    m = (xv >= tauv) & (iv != tgtv)
                    cnt = _scal(plsc.all_reduce_population_count(m))
                    plsc.store_compressed(cval_v.at[pl.ds(oo, SL)], xv,
                                          mask=m)
                    plsc.store_compressed(cidx_v.at[pl.ds(oo, SL)], iv,
                                          mask=m)
                    return oo + cnt

                return lax.fori_loop(0, nsub, sub, o)

            return lax.cond(bms >= taus, hit, lambda o: o, off)

        c = lax.fori_loop(0, v // LN, filt, jnp.int32(0))
        cval_v[pl.ds(c, SL)] = ninf_v
        c16 = (c + SL - 1) // SL

        a0v = jnp.where(iota16 == 0, goldv, ninf_v)
        a0i = jnp.where(iota16 == 0, tgtv, zero_i)

        def ext(k, carry):
            avs = list(carry[:4])
            ais = list(carry[4:])

            def scan(j, bc):
                best, pos = bc
                cv = cval_v[pl.ds(j * SL, SL)]
                mj = jnp.max(cv, axis=0)
                pj = _scal(plsc.all_reduce_ffs(cv == (zero_f + mj)))
                upd = mj > best
                return (jnp.where(upd, mj, best),
                        jnp.where(upd, j * SL + pj, pos))

            best, pos = lax.fori_loop(0, c16, scan, (NINF, jnp.int32(0)))
            posv = zero_i + pos
            bvv = zero_f + best
            biv = plsc.load_gather(cidx_v, [posv])
            plsc.store_scatter(cval_v, [posv], ninf_v, mask=iota16 == 0)
            km = k % SL
            kd = k // SL
            for j in range(4):
                m = (iota16 == km) & (kd == j)
                avs[j] = jnp.where(m, bvv, avs[j])
                ais[j] = jnp.where(m, biv, ais[j])
            return tuple(avs) + tuple(ais)

        accs = lax.fori_loop(
            1, KB, ext,
            (a0v, ninf_v, ninf_v, ninf_v, a0i, zero_i, zero_i, zero_i))
        for j in range(4):
            vals_v[pl.ds(j * SL, SL)] = accs[j]
            ids_v[pl.ds(j * SL, SL)] = accs[4 + j]
        for ghbm, gv in ((e1_hbm, g1_v), (e2_hbm, g2_v)):
            pltpu.async_copy(ghbm.at[ids_v], gpad_v, sem).wait()

            def compact(kk, _, gv=gv):
                for jj in range(d // SL):
                    gv[kk, pl.ds(jj * SL, SL)] = gpad_v[kk, pl.ds(jj * SL, SL)]
                return 0

            lax.fori_loop(0, KB, compact, 0)
        pltpu.sync_copy(vals_v, vals_hbm.at[row])
        pltpu.sync_copy(g1_v, g1_hbm.at[row])
        pltpu.sync_copy(g2_v, g2_hbm.at[row])
        return 0

    lax.fori_loop(0, rpw, do_row, 0)


def _tail_body(mask_smem, vals_ref, g1_ref, g2_ref, mask_ref, out_ref, *, t):
    b = pl.program_id(0)

    @pl.when(b == 0)
    def _():
        out_ref[0, 0] = 0.0

    v = vals_ref[0]
    mx = jnp.max(v, axis=-1, keepdims=True)
    lse = jnp.log(jnp.sum(jnp.exp(v - mx), axis=-1, keepdims=True)) + mx
    node = (v[:, :1] - lse).reshape(1, t)
    msk = mask_ref[0, 0].reshape(1, t)
    node_sum = jnp.sum(node * msk)
    tsum = jnp.sum(msk)

    s1 = g1_ref[0, :t - 1]
    s2 = g2_ref[0, 1:]
    mm = lax.dot_general(s1, s2, (((2,), (2,)), ((0,), (0,))),
                         preferred_element_type=jnp.float32)
    mf = mm.reshape(t - 1, KB * KB)
    mx2 = jnp.max(mf, axis=-1, keepdims=True)
    lse2 = jnp.log(jnp.sum(jnp.exp(mf - mx2), axis=-1, keepdims=True)) + mx2
    edge = (mf[:, :1] - lse2).reshape(1, t - 1)
    edge_sum = jnp.sum(edge * msk[:, 1:])
    nb = pl.num_programs(0)
    out_ref[0, 0] += -(node_sum + edge_sum) / tsum / nb


def kernel(unaries, masks, targets, E1_weight, E2_weight):
    b, t, v = unaries.shape
    d = E1_weight.shape[-1]
    rs = v // LN
    nrows = b * t
    nprog = nrows // TB
    u3 = unaries.reshape(nrows, rs, LN)

    tau3 = pl.pallas_call(
        functools.partial(_tau_body, rs=rs),
        grid=(nprog,),
        in_specs=[pl.BlockSpec((TB, rs, LN), lambda i: (i, 0, 0))],
        out_specs=pl.BlockSpec((1, 1, TB), lambda i: (i, 0, 0)),
        out_shape=jax.ShapeDtypeStruct((nprog, 1, TB), jnp.float32),
    )(u3)
    tau = tau3.reshape(nrows)

    rpw = nrows // NW
    sc_beam = functools.partial(
        pl.kernel,
        out_type=[
            jax.ShapeDtypeStruct((nrows, KB), jnp.float32),
            jax.ShapeDtypeStruct((nrows, KB, d), jnp.float32),
            jax.ShapeDtypeStruct((nrows, KB, d), jnp.float32),
        ],
        mesh=plsc.VectorSubcoreMesh(core_axis_name="c", subcore_axis_name="s",
                                    num_cores=2, num_subcores=16),
        compiler_params=pltpu.CompilerParams(needs_layout_passes=False),
        scratch_types=[
            pltpu.VMEM((v // LN, LN), jnp.float32),
            pltpu.VMEM((v + SL,), jnp.float32),
            pltpu.VMEM((v + SL,), jnp.int32),
            pltpu.VMEM((rpw,), jnp.int32),
            pltpu.VMEM((rpw,), jnp.float32),
            pltpu.VMEM((KB,), jnp.float32),
            pltpu.VMEM((KB,), jnp.int32),
            pltpu.VMEM((KB, d), jnp.float32),
            pltpu.VMEM((KB, d), jnp.float32),
            pltpu.VMEM((KB, LN), jnp.float32),
            pltpu.SemaphoreType.DMA,
        ],
    )(functools.partial(_sc_beam_body, v=v, rpw=rpw, d=d))
    e1p = jnp.pad(E1_weight, ((0, 0), (0, LN - d)))
    e2p = jnp.pad(E2_weight, ((0, 0), (0, LN - d)))
    vals, g1, g2 = sc_beam(u3, targets.astype(jnp.int32).reshape(nrows), tau,
                           e1p, e2p)

    vals = vals.reshape(b, t, KB)
    g1 = g1.reshape(b, t, KB, d)
    g2 = g2.reshape(b, t, KB, d)

    nll = pl.pallas_call(
        functools.partial(_tail_body, t=t),
        grid=(b,),
        in_specs=[
            pl.BlockSpec(memory_space=pltpu.SMEM),
            pl.BlockSpec((1, t, KB), lambda i: (i, 0, 0)),
            pl.BlockSpec((1, t, KB, d), lambda i: (i, 0, 0, 0)),
            pl.BlockSpec((1, t, KB, d), lambda i: (i, 0, 0, 0)),
            pl.BlockSpec((1, 1, t), lambda i: (i, 0, 0)),
        ],
        out_specs=pl.BlockSpec(memory_space=pltpu.SMEM),
        out_shape=jax.ShapeDtypeStruct((1, 1), jnp.float32),
    )(masks.astype(jnp.int32), vals, g1, g2,
      masks.astype(jnp.float32).reshape(b, 1, t))

    return nll[0, 0]
